# Initial kernel scaffold; baseline (speedup 1.0000x reference)
import jax
import jax.numpy as jnp
from jax.experimental import pallas as pl


def kernel(x, edge_index, W1, a_src1, a_dst1, b1, W2, a_src2, a_dst2, b2):
    raise NotImplementedError("write your pallas kernel here")


# speedup vs baseline: 69.7774x; 69.7774x over previous
import functools

import jax
import jax.numpy as jnp
from jax import lax
from jax.experimental import pallas as pl
from jax.experimental.pallas import tpu as pltpu
from jax.experimental.pallas import tpu_sc as plsc

_N = 10000
_NP = 10240
_E = 320000
_H = 8
_D = 64
_AW = 80
_B = 128
_NWORK = 32
_RPW = 80
_EP = _B * _NWORK * _RPW
_NROWS = _EP // _B
_BLK = 512
_GRID = _NP // _BLK

_f32 = jnp.float32


def _sc_edge_body(src_hbm, dst_hbm, alpha_hbm, h_hbm, init_hbm, out_hbm,
                  sidx, didx, abufs, abufd, hbuf, obuf, acc,
                  sem_h, sem_a, sem_b):
    c = lax.axis_index("c")
    s = lax.axis_index("s")
    w = c * 16 + s

    @pl.when(s == 0)
    def _():
        pltpu.sync_copy(init_hbm.at[c], acc)
    plsc.subcore_barrier()

    pltpu.sync_copy(src_hbm.at[pl.ds(w * _RPW, _RPW)], sidx)
    pltpu.sync_copy(dst_hbm.at[pl.ds(w * _RPW, _RPW)], didx)

    def _zero(e, carry):
        obuf[e, pl.ds(64, 16)] = jnp.zeros((16,), _f32)
        return carry
    lax.fori_loop(0, _B, _zero, 0)

    lanes = lax.iota(jnp.int32, 16)
    pats = [64 + 2 * cc + lanes // 8 for cc in range(4)]

    def _block(b, carry):
        cph = pltpu.async_copy(h_hbm.at[sidx.at[b]], hbuf, sem_h)
        cpa = pltpu.async_copy(alpha_hbm.at[sidx.at[b]], abufs, sem_a)
        cpb = pltpu.async_copy(alpha_hbm.at[didx.at[b]], abufd, sem_b)
        cpa.wait()
        cpb.wait()
        for g in range(_B // 16):
            erow = g * 16 + lanes
            for hd in range(_H):
                av = plsc.load_gather(abufs, [erow, jnp.full((16,), hd, jnp.int32)])
                bv = plsc.load_gather(abufd, [erow, jnp.full((16,), 8 + hd, jnp.int32)])
                t = av + bv
                t = jnp.where(t >= 0.0, t, 0.2 * t)
                plsc.store_scatter(obuf, [erow, jnp.full((16,), 64 + hd, jnp.int32)],
                                   jnp.exp(t))
        cph.wait()

        def _edge(e, carry2):
            ev = jnp.full((16,), e, jnp.int32)
            for cc in range(4):
                sexp = plsc.load_gather(obuf, [ev, pats[cc]])
                obuf[e, pl.ds(cc * 16, 16)] = hbuf[e, pl.ds(cc * 16, 16)] * sexp
            return carry2
        lax.fori_loop(0, _B, _edge, 0)

        pltpu.sync_copy(obuf, acc.at[didx.at[b]], add=True)
        return carry

    lax.fori_loop(0, _RPW, _block, 0)

    plsc.subcore_barrier()
    @pl.when(s == 0)
    def _():
        pltpu.sync_copy(acc, out_hbm.at[c])


_sc_edges = pl.kernel(
    _sc_edge_body,
    out_type=jax.ShapeDtypeStruct((2, _NP, _AW), _f32),
    mesh=plsc.VectorSubcoreMesh(core_axis_name="c", subcore_axis_name="s",
                                num_cores=2, num_subcores=16),
    scratch_types=[
        pltpu.VMEM((_RPW, _B), jnp.int32),
        pltpu.VMEM((_RPW, _B), jnp.int32),
        pltpu.VMEM((_B, 16), _f32),
        pltpu.VMEM((_B, 16), _f32),
        pltpu.VMEM((_B, _D), _f32),
        pltpu.VMEM((_B, _AW), _f32),
        pltpu.VMEM_SHARED((_NP, _AW), _f32),
        pltpu.SemaphoreType.DMA,
        pltpu.SemaphoreType.DMA,
        pltpu.SemaphoreType.DMA,
    ],
    compiler_params=pltpu.CompilerParams(use_tc_tiling_on_sc=False, needs_layout_passes=False),
)


def _tc_pre_body(x_ref, w1_ref, asrcm_ref, adstm_ref, r1_ref,
                 h_ref, alpha_ref, init_ref):
    h = jnp.dot(x_ref[:], w1_ref[:], preferred_element_type=_f32)
    asrc = jnp.dot(h, asrcm_ref[:], preferred_element_type=_f32)
    adst = jnp.dot(h, adstm_ref[:], preferred_element_type=_f32)
    alpha_ref[:, 0:8] = asrc
    alpha_ref[:, 8:16] = adst
    t = asrc + adst
    t = jnp.where(t >= 0.0, t, 0.2 * t)
    sv = jnp.exp(t)
    srep = jnp.dot(sv, r1_ref[:], preferred_element_type=_f32)
    init_ref[0, :, 0:64] = h * srep
    init_ref[0, :, 64:72] = sv
    init_ref[0, :, 72:80] = jnp.zeros((_BLK, 8), _f32)
    init_ref[1, :, :] = jnp.zeros((_BLK, _AW), _f32)
    h_ref[:] = h


def _tc_mid_body(acc_ref, b1_ref, w2_ref, asrcm_ref, adstm_ref, r1_ref, r2_ref,
                 h2_ref, alpha_ref, init_ref):
    accs = acc_ref[0] + acc_ref[1]
    num = accs[:, 0:64]
    den = jnp.dot(accs[:, 64:72], r1_ref[:], preferred_element_type=_f32)
    out1 = num / (den + 1e-16) + b1_ref[:]
    h2 = jnp.where(out1 > 0.0, out1, jnp.exp(out1) - 1.0)
    g = jnp.dot(h2, w2_ref[:], preferred_element_type=_f32)
    asrc = jnp.dot(g, asrcm_ref[:], preferred_element_type=_f32)
    adst = jnp.dot(g, adstm_ref[:], preferred_element_type=_f32)
    alpha_ref[:, 0:8] = asrc
    alpha_ref[:, 8:16] = adst
    t = asrc + adst
    t = jnp.where(t >= 0.0, t, 0.2 * t)
    sv = jnp.exp(t)
    srep = jnp.dot(sv, r2_ref[:], preferred_element_type=_f32)
    init_ref[0, :, 0:64] = g * srep
    init_ref[0, :, 64:72] = sv
    init_ref[0, :, 72:80] = jnp.zeros((_BLK, 8), _f32)
    init_ref[1, :, :] = jnp.zeros((_BLK, _AW), _f32)
    h2_ref[:] = g


def _tc_post_body(acc_ref, b2_ref, r2_ref, out_ref):
    accs = acc_ref[0] + acc_ref[1]
    num = accs[:, 0:64]
    den = jnp.dot(accs[:, 64:72], r2_ref[:], preferred_element_type=_f32)
    z = num / (den + 1e-16) + b2_ref[:]
    col = lax.broadcasted_iota(jnp.int32, (_BLK, 64), 1)
    zm = jnp.where(col < 56, z, -1e30)
    m = jnp.max(zm, axis=1, keepdims=True)
    lse = jnp.log(jnp.sum(jnp.exp(zm - m), axis=1, keepdims=True)) + m
    out_ref[:] = (z - lse)[:, 0:56]


def _full(shape):
    return pl.BlockSpec(shape, lambda i: tuple(0 for _ in shape))


_tc_pre = pl.pallas_call(
    _tc_pre_body,
    grid=(_GRID,),
    in_specs=[pl.BlockSpec((_BLK, 128), lambda i: (i, 0)),
              _full((128, 64)), _full((64, 8)), _full((64, 8)), _full((8, 64))],
    out_specs=[pl.BlockSpec((_BLK, 64), lambda i: (i, 0)),
               pl.BlockSpec((_BLK, 16), lambda i: (i, 0)),
               pl.BlockSpec((2, _BLK, _AW), lambda i: (0, i, 0))],
    out_shape=[jax.ShapeDtypeStruct((_NP, 64), _f32),
               jax.ShapeDtypeStruct((_NP, 16), _f32),
               jax.ShapeDtypeStruct((2, _NP, _AW), _f32)],
)

_tc_mid = pl.pallas_call(
    _tc_mid_body,
    grid=(_GRID,),
    in_specs=[pl.BlockSpec((2, _BLK, _AW), lambda i: (0, i, 0)),
              _full((1, 64)), _full((64, 64)), _full((64, 8)), _full((64, 8)),
              _full((8, 64)), _full((8, 64))],
    out_specs=[pl.BlockSpec((_BLK, 64), lambda i: (i, 0)),
               pl.BlockSpec((_BLK, 16), lambda i: (i, 0)),
               pl.BlockSpec((2, _BLK, _AW), lambda i: (0, i, 0))],
    out_shape=[jax.ShapeDtypeStruct((_NP, 64), _f32),
               jax.ShapeDtypeStruct((_NP, 16), _f32),
               jax.ShapeDtypeStruct((2, _NP, _AW), _f32)],
)

_tc_post = pl.pallas_call(
    _tc_post_body,
    grid=(_GRID,),
    in_specs=[pl.BlockSpec((2, _BLK, _AW), lambda i: (0, i, 0)),
              _full((1, 64)), _full((8, 64))],
    out_specs=pl.BlockSpec((_BLK, 56), lambda i: (i, 0)),
    out_shape=jax.ShapeDtypeStruct((_NP, 56), _f32),
)


def kernel(x, edge_index, W1, a_src1, a_dst1, b1, W2, a_src2, a_dst2, b2):
    xp = jnp.pad(x, ((0, _NP - _N), (0, 0)))
    src2d = jnp.pad(edge_index[0], (0, _EP - _E)).reshape(_NROWS, _B)
    dst2d = jnp.pad(edge_index[1], (0, _EP - _E),
                    constant_values=_N).reshape(_NROWS, _B)

    eye = jnp.eye(_H, dtype=_f32)
    asrcm1 = (eye[:, None, :] * a_src1[:, :, None]).reshape(64, _H)
    adstm1 = (eye[:, None, :] * a_dst1[:, :, None]).reshape(64, _H)
    r1 = (eye[:, :, None] * jnp.ones((1, 1, 8), _f32)).reshape(_H, 64)
    asrcm2 = jnp.pad((eye[:, None, :] * a_src2[:, :, None]).reshape(56, _H),
                     ((0, 8), (0, 0)))
    adstm2 = jnp.pad((eye[:, None, :] * a_dst2[:, :, None]).reshape(56, _H),
                     ((0, 8), (0, 0)))
    r2 = jnp.pad((eye[:, :, None] * jnp.ones((1, 1, 7), _f32)).reshape(_H, 56),
                 ((0, 0), (0, 8)))
    w2p = jnp.pad(W2, ((0, 0), (0, 8)))
    b1r = b1.reshape(1, 64)
    b2p = jnp.pad(b2, (0, 8)).reshape(1, 64)

    h1, alpha1, init1 = _tc_pre(xp, W1, asrcm1, adstm1, r1)
    acc1 = _sc_edges(src2d, dst2d, alpha1, h1, init1)
    h2, alpha2, init2 = _tc_mid(acc1, b1r, w2p, asrcm2, adstm2, r1, r2)
    acc2 = _sc_edges(src2d, dst2d, alpha2, h2, init2)
    outp = _tc_post(acc2, b2p, r2)
    return outp[:_N]


# speedup vs baseline: 99.0140x; 1.4190x over previous
import functools

import jax
import jax.numpy as jnp
from jax import lax
from jax.experimental import pallas as pl
from jax.experimental.pallas import tpu as pltpu
from jax.experimental.pallas import tpu_sc as plsc

_N = 10000
_NP = 10240
_E = 320000
_H = 8
_D = 64
_AW = 80
_B = 128
_NWORK = 32
_RPW = 80
_EP = _B * _NWORK * _RPW
_NROWS = _EP // _B
_BLK = 512
_GRID = _NP // _BLK

_f32 = jnp.float32


def _sc_edge_body(src_hbm, dst_hbm, alpha_hbm, h_hbm, init_hbm, out_hbm,
                  sidx, didx, abufs, abufd, hbuf, obuf, acc,
                  sem_g0, sem_g1, sem_s0, sem_s1):
    c = lax.axis_index("c")
    s = lax.axis_index("s")
    w = c * 16 + s

    @pl.when(s == 0)
    def _():
        pltpu.sync_copy(init_hbm.at[c], acc)
    plsc.subcore_barrier()

    pltpu.sync_copy(src_hbm.at[pl.ds(w * _RPW, _RPW)], sidx)
    pltpu.sync_copy(dst_hbm.at[pl.ds(w * _RPW, _RPW)], didx)

    def _zero(e, carry):
        obuf[0, e, pl.ds(64, 16)] = jnp.zeros((16,), _f32)
        obuf[1, e, pl.ds(64, 16)] = jnp.zeros((16,), _f32)
        return carry
    lax.fori_loop(0, _B, _zero, 0)

    lanes = lax.iota(jnp.int32, 16)
    pats = [64 + 2 * cc + lanes // 8 for cc in range(4)]
    sem_g = (sem_g0, sem_g1)
    sem_s = (sem_s0, sem_s1)

    def _issue_gathers(p, b):
        pltpu.async_copy(h_hbm.at[sidx.at[b]], hbuf.at[p], sem_g[p])
        pltpu.async_copy(alpha_hbm.at[sidx.at[b]], abufs.at[p], sem_g[p])
        pltpu.async_copy(alpha_hbm.at[didx.at[b]], abufd.at[p], sem_g[p])

    def _wait_gathers(p, b):
        pltpu.make_async_copy(h_hbm.at[sidx.at[b]], hbuf.at[p], sem_g[p]).wait()
        pltpu.make_async_copy(alpha_hbm.at[sidx.at[b]], abufs.at[p], sem_g[p]).wait()
        pltpu.make_async_copy(alpha_hbm.at[didx.at[b]], abufd.at[p], sem_g[p]).wait()

    _issue_gathers(0, 0)

    def _round(r, carry):
        for p in range(2):
            b = 2 * r + p
            _wait_gathers(p, b)

            @pl.when(b + 1 < _RPW)
            def _():
                _issue_gathers(1 - p, b + 1)

            @pl.when(b >= 2)
            def _():
                pltpu.make_async_copy(obuf.at[p], acc.at[didx.at[b]],
                                      sem_s[p]).wait()

            for g in range(_B // 16):
                erow = g * 16 + lanes
                for hd in range(_H):
                    av = plsc.load_gather(abufs.at[p],
                                          [erow, jnp.full((16,), hd, jnp.int32)])
                    bv = plsc.load_gather(abufd.at[p],
                                          [erow, jnp.full((16,), 8 + hd, jnp.int32)])
                    t = av + bv
                    t = jnp.where(t >= 0.0, t, 0.2 * t)
                    plsc.store_scatter(obuf.at[p],
                                       [erow, jnp.full((16,), 64 + hd, jnp.int32)],
                                       jnp.exp(t))

            def _edge(e, carry2):
                ev = jnp.full((16,), e, jnp.int32)
                for cc in range(4):
                    sexp = plsc.load_gather(obuf.at[p], [ev, pats[cc]])
                    obuf[p, e, pl.ds(cc * 16, 16)] = (
                        hbuf[p, e, pl.ds(cc * 16, 16)] * sexp)
                return carry2
            lax.fori_loop(0, _B, _edge, 0)

            pltpu.async_copy(obuf.at[p], acc.at[didx.at[b]], sem_s[p], add=True)
        return carry

    lax.fori_loop(0, _RPW // 2, _round, 0)

    pltpu.make_async_copy(obuf.at[0], acc.at[didx.at[_RPW - 2]], sem_s[0]).wait()
    pltpu.make_async_copy(obuf.at[1], acc.at[didx.at[_RPW - 1]], sem_s[1]).wait()

    plsc.subcore_barrier()
    @pl.when(s == 0)
    def _():
        pltpu.sync_copy(acc, out_hbm.at[c])


_sc_edges = pl.kernel(
    _sc_edge_body,
    out_type=jax.ShapeDtypeStruct((2, _NP, _AW), _f32),
    mesh=plsc.VectorSubcoreMesh(core_axis_name="c", subcore_axis_name="s",
                                num_cores=2, num_subcores=16),
    scratch_types=[
        pltpu.VMEM((_RPW, _B), jnp.int32),
        pltpu.VMEM((_RPW, _B), jnp.int32),
        pltpu.VMEM((2, _B, 16), _f32),
        pltpu.VMEM((2, _B, 16), _f32),
        pltpu.VMEM((2, _B, _D), _f32),
        pltpu.VMEM((2, _B, _AW), _f32),
        pltpu.VMEM_SHARED((_NP, _AW), _f32),
        pltpu.SemaphoreType.DMA,
        pltpu.SemaphoreType.DMA,
        pltpu.SemaphoreType.DMA,
        pltpu.SemaphoreType.DMA,
    ],
    compiler_params=pltpu.CompilerParams(use_tc_tiling_on_sc=False, needs_layout_passes=False),
)


def _tc_pre_body(x_ref, w1_ref, asrcm_ref, adstm_ref, r1_ref,
                 h_ref, alpha_ref, init_ref):
    h = jnp.dot(x_ref[:], w1_ref[:], preferred_element_type=_f32)
    asrc = jnp.dot(h, asrcm_ref[:], preferred_element_type=_f32)
    adst = jnp.dot(h, adstm_ref[:], preferred_element_type=_f32)
    alpha_ref[:, 0:8] = asrc
    alpha_ref[:, 8:16] = adst
    t = asrc + adst
    t = jnp.where(t >= 0.0, t, 0.2 * t)
    sv = jnp.exp(t)
    srep = jnp.dot(sv, r1_ref[:], preferred_element_type=_f32)
    init_ref[0, :, 0:64] = h * srep
    init_ref[0, :, 64:72] = sv
    init_ref[0, :, 72:80] = jnp.zeros((_BLK, 8), _f32)
    init_ref[1, :, :] = jnp.zeros((_BLK, _AW), _f32)
    h_ref[:] = h


def _tc_mid_body(acc_ref, b1_ref, w2_ref, asrcm_ref, adstm_ref, r1_ref, r2_ref,
                 h2_ref, alpha_ref, init_ref):
    accs = acc_ref[0] + acc_ref[1]
    num = accs[:, 0:64]
    den = jnp.dot(accs[:, 64:72], r1_ref[:], preferred_element_type=_f32)
    out1 = num / (den + 1e-16) + b1_ref[:]
    h2 = jnp.where(out1 > 0.0, out1, jnp.exp(out1) - 1.0)
    g = jnp.dot(h2, w2_ref[:], preferred_element_type=_f32)
    asrc = jnp.dot(g, asrcm_ref[:], preferred_element_type=_f32)
    adst = jnp.dot(g, adstm_ref[:], preferred_element_type=_f32)
    alpha_ref[:, 0:8] = asrc
    alpha_ref[:, 8:16] = adst
    t = asrc + adst
    t = jnp.where(t >= 0.0, t, 0.2 * t)
    sv = jnp.exp(t)
    srep = jnp.dot(sv, r2_ref[:], preferred_element_type=_f32)
    init_ref[0, :, 0:64] = g * srep
    init_ref[0, :, 64:72] = sv
    init_ref[0, :, 72:80] = jnp.zeros((_BLK, 8), _f32)
    init_ref[1, :, :] = jnp.zeros((_BLK, _AW), _f32)
    h2_ref[:] = g


def _tc_post_body(acc_ref, b2_ref, r2_ref, out_ref):
    accs = acc_ref[0] + acc_ref[1]
    num = accs[:, 0:64]
    den = jnp.dot(accs[:, 64:72], r2_ref[:], preferred_element_type=_f32)
    z = num / (den + 1e-16) + b2_ref[:]
    col = lax.broadcasted_iota(jnp.int32, (_BLK, 64), 1)
    zm = jnp.where(col < 56, z, -1e30)
    m = jnp.max(zm, axis=1, keepdims=True)
    lse = jnp.log(jnp.sum(jnp.exp(zm - m), axis=1, keepdims=True)) + m
    out_ref[:] = (z - lse)[:, 0:56]


def _full(shape):
    return pl.BlockSpec(shape, lambda i: tuple(0 for _ in shape))


_tc_pre = pl.pallas_call(
    _tc_pre_body,
    grid=(_GRID,),
    in_specs=[pl.BlockSpec((_BLK, 128), lambda i: (i, 0)),
              _full((128, 64)), _full((64, 8)), _full((64, 8)), _full((8, 64))],
    out_specs=[pl.BlockSpec((_BLK, 64), lambda i: (i, 0)),
               pl.BlockSpec((_BLK, 16), lambda i: (i, 0)),
               pl.BlockSpec((2, _BLK, _AW), lambda i: (0, i, 0))],
    out_shape=[jax.ShapeDtypeStruct((_NP, 64), _f32),
               jax.ShapeDtypeStruct((_NP, 16), _f32),
               jax.ShapeDtypeStruct((2, _NP, _AW), _f32)],
)

_tc_mid = pl.pallas_call(
    _tc_mid_body,
    grid=(_GRID,),
    in_specs=[pl.BlockSpec((2, _BLK, _AW), lambda i: (0, i, 0)),
              _full((1, 64)), _full((64, 64)), _full((64, 8)), _full((64, 8)),
              _full((8, 64)), _full((8, 64))],
    out_specs=[pl.BlockSpec((_BLK, 64), lambda i: (i, 0)),
               pl.BlockSpec((_BLK, 16), lambda i: (i, 0)),
               pl.BlockSpec((2, _BLK, _AW), lambda i: (0, i, 0))],
    out_shape=[jax.ShapeDtypeStruct((_NP, 64), _f32),
               jax.ShapeDtypeStruct((_NP, 16), _f32),
               jax.ShapeDtypeStruct((2, _NP, _AW), _f32)],
)

_tc_post = pl.pallas_call(
    _tc_post_body,
    grid=(_GRID,),
    in_specs=[pl.BlockSpec((2, _BLK, _AW), lambda i: (0, i, 0)),
              _full((1, 64)), _full((8, 64))],
    out_specs=pl.BlockSpec((_BLK, 56), lambda i: (i, 0)),
    out_shape=jax.ShapeDtypeStruct((_NP, 56), _f32),
)


def kernel(x, edge_index, W1, a_src1, a_dst1, b1, W2, a_src2, a_dst2, b2):
    xp = jnp.pad(x, ((0, _NP - _N), (0, 0)))
    src2d = jnp.pad(edge_index[0], (0, _EP - _E)).reshape(_NROWS, _B)
    dst2d = jnp.pad(edge_index[1], (0, _EP - _E),
                    constant_values=_N).reshape(_NROWS, _B)

    eye = jnp.eye(_H, dtype=_f32)
    asrcm1 = (eye[:, None, :] * a_src1[:, :, None]).reshape(64, _H)
    adstm1 = (eye[:, None, :] * a_dst1[:, :, None]).reshape(64, _H)
    r1 = (eye[:, :, None] * jnp.ones((1, 1, 8), _f32)).reshape(_H, 64)
    asrcm2 = jnp.pad((eye[:, None, :] * a_src2[:, :, None]).reshape(56, _H),
                     ((0, 8), (0, 0)))
    adstm2 = jnp.pad((eye[:, None, :] * a_dst2[:, :, None]).reshape(56, _H),
                     ((0, 8), (0, 0)))
    r2 = jnp.pad((eye[:, :, None] * jnp.ones((1, 1, 7), _f32)).reshape(_H, 56),
                 ((0, 0), (0, 8)))
    w2p = jnp.pad(W2, ((0, 0), (0, 8)))
    b1r = b1.reshape(1, 64)
    b2p = jnp.pad(b2, (0, 8)).reshape(1, 64)

    h1, alpha1, init1 = _tc_pre(xp, W1, asrcm1, adstm1, r1)
    acc1 = _sc_edges(src2d, dst2d, alpha1, h1, init1)
    h2, alpha2, init2 = _tc_mid(acc1, b1r, w2p, asrcm2, adstm2, r1, r2)
    acc2 = _sc_edges(src2d, dst2d, alpha2, h2, init2)
    outp = _tc_post(acc2, b2p, r2)
    return outp[:_N]


# speedup vs baseline: 115.0788x; 1.1622x over previous
import functools

import jax
import jax.numpy as jnp
from jax import lax
from jax.experimental import pallas as pl
from jax.experimental.pallas import tpu as pltpu
from jax.experimental.pallas import tpu_sc as plsc

_N = 10000
_NP = 10240
_E = 320000
_H = 8
_D = 64
_AW = 80
_B = 128
_NWORK = 32
_RPW = 80
_EP = _B * _NWORK * _RPW
_NROWS = _EP // _B
_BLK = 512
_GRID = _NP // _BLK

_f32 = jnp.float32


def _sc_edge_body(src_hbm, dst_hbm, alpha_hbm, h_hbm, init_hbm, out_hbm,
                  sidx, didx, abufs, abufd, hbuf, obuf, acc,
                  sem_g0, sem_g1, sem_s0, sem_s1):
    c = lax.axis_index("c")
    s = lax.axis_index("s")
    w = c * 16 + s

    @pl.when(s == 0)
    def _():
        pltpu.sync_copy(init_hbm.at[c], acc)
    plsc.subcore_barrier()

    pltpu.sync_copy(src_hbm.at[pl.ds(w * _RPW, _RPW)], sidx)
    pltpu.sync_copy(dst_hbm.at[pl.ds(w * _RPW, _RPW)], didx)

    def _zero(e, carry):
        obuf[0, e, pl.ds(64, 16)] = jnp.zeros((16,), _f32)
        obuf[1, e, pl.ds(64, 16)] = jnp.zeros((16,), _f32)
        return carry
    lax.fori_loop(0, _B, _zero, 0)

    lanes = lax.iota(jnp.int32, 16)
    pats = [64 + 2 * cc + lanes // 8 for cc in range(4)]
    sem_g = (sem_g0, sem_g1)
    sem_s = (sem_s0, sem_s1)

    def _issue_gathers(p, b):
        pltpu.async_copy(h_hbm.at[sidx.at[b]], hbuf.at[p], sem_g[p])
        pltpu.async_copy(alpha_hbm.at[sidx.at[b]], abufs.at[p], sem_g[p])
        pltpu.async_copy(alpha_hbm.at[didx.at[b]], abufd.at[p], sem_g[p])

    def _wait_gathers(p, b):
        pltpu.make_async_copy(h_hbm.at[sidx.at[b]], hbuf.at[p], sem_g[p]).wait()
        pltpu.make_async_copy(alpha_hbm.at[sidx.at[b]], abufs.at[p], sem_g[p]).wait()
        pltpu.make_async_copy(alpha_hbm.at[didx.at[b]], abufd.at[p], sem_g[p]).wait()

    _issue_gathers(0, 0)

    def _round(r, carry):
        for p in range(2):
            b = 2 * r + p
            _wait_gathers(p, b)

            @pl.when(b + 1 < _RPW)
            def _():
                _issue_gathers(1 - p, b + 1)

            @pl.when(b >= 2)
            def _():
                pltpu.make_async_copy(obuf.at[p], acc.at[didx.at[b]],
                                      sem_s[p]).wait()

            for g in range(_B // 16):
                erow = g * 16 + lanes
                for hd in range(_H):
                    av = plsc.load_gather(abufs.at[p],
                                          [erow, jnp.full((16,), hd, jnp.int32)])
                    bv = plsc.load_gather(abufd.at[p],
                                          [erow, jnp.full((16,), 8 + hd, jnp.int32)])
                    t = av + bv
                    t = jnp.where(t >= 0.0, t, 0.2 * t)
                    plsc.store_scatter(obuf.at[p],
                                       [erow, jnp.full((16,), 64 + hd, jnp.int32)],
                                       jnp.exp(t))

            @plsc.parallel_loop(0, _B, unroll=4)
            def _edge(e):
                srow = obuf[p, e, pl.ds(64, 16)]
                for cc in range(4):
                    sexp = srow[2 * cc + lanes // 8]
                    obuf[p, e, pl.ds(cc * 16, 16)] = (
                        hbuf[p, e, pl.ds(cc * 16, 16)] * sexp)

            pltpu.async_copy(obuf.at[p], acc.at[didx.at[b]], sem_s[p], add=True)
        return carry

    lax.fori_loop(0, _RPW // 2, _round, 0)

    pltpu.make_async_copy(obuf.at[0], acc.at[didx.at[_RPW - 2]], sem_s[0]).wait()
    pltpu.make_async_copy(obuf.at[1], acc.at[didx.at[_RPW - 1]], sem_s[1]).wait()

    plsc.subcore_barrier()
    @pl.when(s == 0)
    def _():
        pltpu.sync_copy(acc, out_hbm.at[c])


_sc_edges = pl.kernel(
    _sc_edge_body,
    out_type=jax.ShapeDtypeStruct((2, _NP, _AW), _f32),
    mesh=plsc.VectorSubcoreMesh(core_axis_name="c", subcore_axis_name="s",
                                num_cores=2, num_subcores=16),
    scratch_types=[
        pltpu.VMEM((_RPW, _B), jnp.int32),
        pltpu.VMEM((_RPW, _B), jnp.int32),
        pltpu.VMEM((2, _B, 16), _f32),
        pltpu.VMEM((2, _B, 16), _f32),
        pltpu.VMEM((2, _B, _D), _f32),
        pltpu.VMEM((2, _B, _AW), _f32),
        pltpu.VMEM_SHARED((_NP, _AW), _f32),
        pltpu.SemaphoreType.DMA,
        pltpu.SemaphoreType.DMA,
        pltpu.SemaphoreType.DMA,
        pltpu.SemaphoreType.DMA,
    ],
    compiler_params=pltpu.CompilerParams(use_tc_tiling_on_sc=False, needs_layout_passes=False),
)


def _tc_pre_body(x_ref, w1_ref, asrcm_ref, adstm_ref, r1_ref,
                 h_ref, alpha_ref, init_ref):
    h = jnp.dot(x_ref[:], w1_ref[:], preferred_element_type=_f32)
    asrc = jnp.dot(h, asrcm_ref[:], preferred_element_type=_f32)
    adst = jnp.dot(h, adstm_ref[:], preferred_element_type=_f32)
    alpha_ref[:, 0:8] = asrc
    alpha_ref[:, 8:16] = adst
    t = asrc + adst
    t = jnp.where(t >= 0.0, t, 0.2 * t)
    sv = jnp.exp(t)
    srep = jnp.dot(sv, r1_ref[:], preferred_element_type=_f32)
    init_ref[0, :, 0:64] = h * srep
    init_ref[0, :, 64:72] = sv
    init_ref[0, :, 72:80] = jnp.zeros((_BLK, 8), _f32)
    init_ref[1, :, :] = jnp.zeros((_BLK, _AW), _f32)
    h_ref[:] = h


def _tc_mid_body(acc_ref, b1_ref, w2_ref, asrcm_ref, adstm_ref, r1_ref, r2_ref,
                 h2_ref, alpha_ref, init_ref):
    accs = acc_ref[0] + acc_ref[1]
    num = accs[:, 0:64]
    den = jnp.dot(accs[:, 64:72], r1_ref[:], preferred_element_type=_f32)
    out1 = num / (den + 1e-16) + b1_ref[:]
    h2 = jnp.where(out1 > 0.0, out1, jnp.exp(out1) - 1.0)
    g = jnp.dot(h2, w2_ref[:], preferred_element_type=_f32)
    asrc = jnp.dot(g, asrcm_ref[:], preferred_element_type=_f32)
    adst = jnp.dot(g, adstm_ref[:], preferred_element_type=_f32)
    alpha_ref[:, 0:8] = asrc
    alpha_ref[:, 8:16] = adst
    t = asrc + adst
    t = jnp.where(t >= 0.0, t, 0.2 * t)
    sv = jnp.exp(t)
    srep = jnp.dot(sv, r2_ref[:], preferred_element_type=_f32)
    init_ref[0, :, 0:64] = g * srep
    init_ref[0, :, 64:72] = sv
    init_ref[0, :, 72:80] = jnp.zeros((_BLK, 8), _f32)
    init_ref[1, :, :] = jnp.zeros((_BLK, _AW), _f32)
    h2_ref[:] = g


def _tc_post_body(acc_ref, b2_ref, r2_ref, out_ref):
    accs = acc_ref[0] + acc_ref[1]
    num = accs[:, 0:64]
    den = jnp.dot(accs[:, 64:72], r2_ref[:], preferred_element_type=_f32)
    z = num / (den + 1e-16) + b2_ref[:]
    col = lax.broadcasted_iota(jnp.int32, (_BLK, 64), 1)
    zm = jnp.where(col < 56, z, -1e30)
    m = jnp.max(zm, axis=1, keepdims=True)
    lse = jnp.log(jnp.sum(jnp.exp(zm - m), axis=1, keepdims=True)) + m
    out_ref[:] = (z - lse)[:, 0:56]


def _full(shape):
    return pl.BlockSpec(shape, lambda i: tuple(0 for _ in shape))


_tc_pre = pl.pallas_call(
    _tc_pre_body,
    grid=(_GRID,),
    in_specs=[pl.BlockSpec((_BLK, 128), lambda i: (i, 0)),
              _full((128, 64)), _full((64, 8)), _full((64, 8)), _full((8, 64))],
    out_specs=[pl.BlockSpec((_BLK, 64), lambda i: (i, 0)),
               pl.BlockSpec((_BLK, 16), lambda i: (i, 0)),
               pl.BlockSpec((2, _BLK, _AW), lambda i: (0, i, 0))],
    out_shape=[jax.ShapeDtypeStruct((_NP, 64), _f32),
               jax.ShapeDtypeStruct((_NP, 16), _f32),
               jax.ShapeDtypeStruct((2, _NP, _AW), _f32)],
)

_tc_mid = pl.pallas_call(
    _tc_mid_body,
    grid=(_GRID,),
    in_specs=[pl.BlockSpec((2, _BLK, _AW), lambda i: (0, i, 0)),
              _full((1, 64)), _full((64, 64)), _full((64, 8)), _full((64, 8)),
              _full((8, 64)), _full((8, 64))],
    out_specs=[pl.BlockSpec((_BLK, 64), lambda i: (i, 0)),
               pl.BlockSpec((_BLK, 16), lambda i: (i, 0)),
               pl.BlockSpec((2, _BLK, _AW), lambda i: (0, i, 0))],
    out_shape=[jax.ShapeDtypeStruct((_NP, 64), _f32),
               jax.ShapeDtypeStruct((_NP, 16), _f32),
               jax.ShapeDtypeStruct((2, _NP, _AW), _f32)],
)

_tc_post = pl.pallas_call(
    _tc_post_body,
    grid=(_GRID,),
    in_specs=[pl.BlockSpec((2, _BLK, _AW), lambda i: (0, i, 0)),
              _full((1, 64)), _full((8, 64))],
    out_specs=pl.BlockSpec((_BLK, 56), lambda i: (i, 0)),
    out_shape=jax.ShapeDtypeStruct((_NP, 56), _f32),
)


def kernel(x, edge_index, W1, a_src1, a_dst1, b1, W2, a_src2, a_dst2, b2):
    xp = jnp.pad(x, ((0, _NP - _N), (0, 0)))
    src2d = jnp.pad(edge_index[0], (0, _EP - _E)).reshape(_NROWS, _B)
    dst2d = jnp.pad(edge_index[1], (0, _EP - _E),
                    constant_values=_N).reshape(_NROWS, _B)

    eye = jnp.eye(_H, dtype=_f32)
    asrcm1 = (eye[:, None, :] * a_src1[:, :, None]).reshape(64, _H)
    adstm1 = (eye[:, None, :] * a_dst1[:, :, None]).reshape(64, _H)
    r1 = (eye[:, :, None] * jnp.ones((1, 1, 8), _f32)).reshape(_H, 64)
    asrcm2 = jnp.pad((eye[:, None, :] * a_src2[:, :, None]).reshape(56, _H),
                     ((0, 8), (0, 0)))
    adstm2 = jnp.pad((eye[:, None, :] * a_dst2[:, :, None]).reshape(56, _H),
                     ((0, 8), (0, 0)))
    r2 = jnp.pad((eye[:, :, None] * jnp.ones((1, 1, 7), _f32)).reshape(_H, 56),
                 ((0, 0), (0, 8)))
    w2p = jnp.pad(W2, ((0, 0), (0, 8)))
    b1r = b1.reshape(1, 64)
    b2p = jnp.pad(b2, (0, 8)).reshape(1, 64)

    h1, alpha1, init1 = _tc_pre(xp, W1, asrcm1, adstm1, r1)
    acc1 = _sc_edges(src2d, dst2d, alpha1, h1, init1)
    h2, alpha2, init2 = _tc_mid(acc1, b1r, w2p, asrcm2, adstm2, r1, r2)
    acc2 = _sc_edges(src2d, dst2d, alpha2, h2, init2)
    outp = _tc_post(acc2, b2p, r2)
    return outp[:_N]
